# baseline (speedup 1.0000x reference)
import jax
import jax.numpy as jnp
from jax.experimental import pallas as pl
from jax.experimental.pallas import tpu as pltpu

_N = 100000
_D = 128
_K = 10
_SPEED = 2.0

_BLK = 20096
_GRID = (_N + _BLK - 1) // _BLK
_SROWS = _GRID * (_BLK // 128)


def _dist_topk_kernel(x_ref, y_ref, data_ref, out_ref, d2_ref):
    i = pl.program_id(0)
    xv = x_ref[...]
    blk = data_ref[...]
    diff = blk - xv
    sq = diff * diff
    d2 = jnp.sum(sq.reshape(_BLK // 128, 128, 128), axis=2)

    g = jax.lax.broadcasted_iota(jnp.int32, d2.shape, 0)
    r = jax.lax.broadcasted_iota(jnp.int32, d2.shape, 1)
    row = i * _BLK + g * 128 + r
    d2 = jnp.where(row < _N, d2, jnp.inf)
    d2_ref[pl.ds(i * (_BLK // 128), _BLK // 128), :] = d2

    @pl.when(i == _GRID - 1)
    def _finalize():
        s = d2_ref[...]
        total = jnp.float32(0.0)
        rem = jnp.float32(_K)
        for _ in range(_K):
            m = jnp.min(s)
            eq = s == m
            c = jnp.sum(jnp.where(eq, 1.0, 0.0))
            take = jnp.minimum(c, rem)
            total = total + jnp.where(take > 0, take * jnp.sqrt(m), 0.0)
            rem = rem - take
            s = jnp.where(eq, jnp.inf, s)
        xy = x_ref[...] - y_ref[...]
        geo = jnp.sqrt(jnp.sum(xy * xy)) / jnp.float32(_SPEED)
        out_ref[...] = (geo + total / jnp.float32(_K)).reshape(1, 1)


@jax.jit
def kernel(x, y, data):
    x2 = x.reshape(1, _D)
    y2 = y.reshape(1, _D)
    out = pl.pallas_call(
        _dist_topk_kernel,
        grid=(_GRID,),
        in_specs=[
            pl.BlockSpec((1, _D), lambda i: (0, 0)),
            pl.BlockSpec((1, _D), lambda i: (0, 0)),
            pl.BlockSpec((_BLK, _D), lambda i: (i, 0)),
        ],
        out_specs=pl.BlockSpec((1, 1), lambda i: (0, 0)),
        out_shape=jax.ShapeDtypeStruct((1, 1), jnp.float32),
        scratch_shapes=[pltpu.VMEM((_SROWS, 128), jnp.float32)],
    )(x2, y2, data)
    return out[0, 0]


# speedup vs baseline: 1.0157x; 1.0157x over previous
import jax
import jax.numpy as jnp
from jax.experimental import pallas as pl
from jax.experimental.pallas import tpu as pltpu

_N = 100000
_D = 128
_K = 10
_SPEED = 2.0

_BLK = 25088
_GRID = (_N + _BLK - 1) // _BLK
_SROWS = _GRID * (_BLK // 128)


def _dist_topk_kernel(x_ref, y_ref, data_ref, out_ref, d2_ref):
    i = pl.program_id(0)
    xv = x_ref[...]
    blk = data_ref[...]
    diff = blk - xv
    sq = diff * diff
    d2 = jnp.sum(sq.reshape(_BLK // 128, 128, 128), axis=2)

    g = jax.lax.broadcasted_iota(jnp.int32, d2.shape, 0)
    r = jax.lax.broadcasted_iota(jnp.int32, d2.shape, 1)
    row = i * _BLK + g * 128 + r
    d2 = jnp.where(row < _N, d2, jnp.inf)
    d2_ref[pl.ds(i * (_BLK // 128), _BLK // 128), :] = d2

    @pl.when(i == _GRID - 1)
    def _finalize():
        s = d2_ref[...]
        total = jnp.float32(0.0)
        rem = jnp.float32(_K)
        for _ in range(_K):
            m = jnp.min(s)
            eq = s == m
            c = jnp.sum(jnp.where(eq, 1.0, 0.0))
            take = jnp.minimum(c, rem)
            total = total + jnp.where(take > 0, take * jnp.sqrt(m), 0.0)
            rem = rem - take
            s = jnp.where(eq, jnp.inf, s)
        xy = x_ref[...] - y_ref[...]
        geo = jnp.sqrt(jnp.sum(xy * xy)) / jnp.float32(_SPEED)
        out_ref[...] = (geo + total / jnp.float32(_K)).reshape(1, 1)


@jax.jit
def kernel(x, y, data):
    x2 = x.reshape(1, _D)
    y2 = y.reshape(1, _D)
    out = pl.pallas_call(
        _dist_topk_kernel,
        grid=(_GRID,),
        in_specs=[
            pl.BlockSpec((1, _D), lambda i: (0, 0)),
            pl.BlockSpec((1, _D), lambda i: (0, 0)),
            pl.BlockSpec((_BLK, _D), lambda i: (i, 0)),
        ],
        out_specs=pl.BlockSpec((1, 1), lambda i: (0, 0)),
        out_shape=jax.ShapeDtypeStruct((1, 1), jnp.float32),
        scratch_shapes=[pltpu.VMEM((_SROWS, 128), jnp.float32)],
    )(x2, y2, data)
    return out[0, 0]
